# baseline (speedup 1.0000x reference)
import jax
import jax.numpy as jnp
from jax.experimental import pallas as pl
from jax.experimental.pallas import tpu as pltpu

_B, _S, _D = 1, 2048, 768
_E, _K = 8, 2
_H = _D * 2
_CF = 1.25
_AUX = 0.01
_T = _B * _S
_C = max(1, int(_T * _K / _E * _CF))
_HSPLIT = 2
_HB = _H // _HSPLIT


def _moe_kernel(x_ref, wr_ref, t_ref, wg_ref, wu_ref, wd_ref,
                o_ref, aux_ref, ent_ref, ovf_ref, pos_scr, cf_scr,
                buf_scr, ob_scr):
    e = pl.program_id(0)
    h = pl.program_id(1)
    f32 = jnp.float32

    @pl.when((e == 0) & (h == 0))
    def _router():
        xf = x_ref[...]
        wr = wr_ref[...]
        t = jnp.clip(t_ref[0, 0], 0.1, 5.0)
        logits = jax.lax.dot_general(
            xf, wr, (((1,), (1,)), ((), ())),
            preferred_element_type=f32) / t
        m = jnp.max(logits, axis=1, keepdims=True)
        ex = jnp.exp(logits - m)
        probs = ex / jnp.sum(ex, axis=1, keepdims=True)

        eio = jax.lax.broadcasted_iota(jnp.int32, (_T, _E), 1)
        m0 = jnp.max(probs, axis=1, keepdims=True)
        i0 = jnp.min(jnp.where(probs == m0, eio, _E), axis=1, keepdims=True)
        p2 = jnp.where(eio == i0, -jnp.inf, probs)
        m1 = jnp.max(p2, axis=1, keepdims=True)
        i1 = jnp.min(jnp.where(p2 == m1, eio, _E), axis=1, keepdims=True)
        mask = ((eio == i0) | (eio == i1)).astype(f32)

        tio = jax.lax.broadcasted_iota(jnp.int32, (_T, _E), 0)
        cum = mask
        s = 1
        while s < _T:
            sh = pltpu.roll(cum, s, axis=0)
            cum = cum + jnp.where(tio >= s, sh, 0.0)
            s *= 2

        keep = mask * (cum <= _C).astype(f32)
        pos_scr[...] = jnp.where(keep > 0, cum - 1.0, -1.0)
        cf_scr[...] = probs * keep

        importance = jnp.sum(probs, axis=0, keepdims=True) / _T
        load = jnp.sum(mask, axis=0, keepdims=True) / (_T + 1e-06)
        aux_ref[...] = jnp.sum(importance * load, axis=1,
                               keepdims=True) * _E * _AUX
        plogp = probs * jnp.log(jnp.clip(probs, 1e-08))
        ent = -jnp.sum(plogp, axis=1, keepdims=True)
        ent_ref[...] = jnp.sum(ent, axis=0, keepdims=True) / _T * 0.01
        nm = jnp.sum(mask, axis=0, keepdims=True)
        nk = jnp.sum(keep, axis=0, keepdims=True)
        tot = jnp.sum(nm, axis=1, keepdims=True)
        drop = jnp.sum(nm - nk, axis=1, keepdims=True)
        ovf_ref[...] = drop / jnp.maximum(tot, 1.0)

    lio = jax.lax.broadcasted_iota(jnp.int32, (_T, _E), 1)
    ohm = (lio == e).astype(f32)
    pos_col = jnp.sum(pos_scr[...] * ohm, axis=1, keepdims=True)
    cf_col = jnp.sum(cf_scr[...] * ohm, axis=1, keepdims=True)

    cj = jax.lax.broadcasted_iota(jnp.int32, (_T, _C), 1)
    ptj = jnp.broadcast_to(pos_col, (_T, _C)).astype(jnp.int32)
    PT = (ptj == cj).astype(f32)

    @pl.when(h == 0)
    def _dispatch():
        buf_scr[...] = jax.lax.dot_general(
            PT, x_ref[...], (((0,), (0,)), ((), ())),
            preferred_element_type=f32)

    buf = buf_scr[...]
    wg = wg_ref[0]
    wu = wu_ref[0]
    wd = wd_ref[0]
    g = jax.lax.dot_general(buf, wg, (((1,), (1,)), ((), ())),
                            preferred_element_type=f32)
    u = jax.lax.dot_general(buf, wu, (((1,), (1,)), ((), ())),
                            preferred_element_type=f32)
    hid = (u * jax.nn.sigmoid(u)) * g
    ob = jax.lax.dot_general(hid, wd, (((1,), (1,)), ((), ())),
                             preferred_element_type=f32)

    @pl.when(h == 0)
    def _():
        ob_scr[...] = ob

    @pl.when(h != 0)
    def _():
        ob_scr[...] += ob

    @pl.when(h == _HSPLIT - 1)
    def _combine():
        comb = jax.lax.dot_general(PT, ob_scr[...], (((1,), (0,)), ((), ())),
                                   preferred_element_type=f32)
        contrib = comb * cf_col

        @pl.when(e == 0)
        def _():
            o_ref[...] = contrib

        @pl.when(e != 0)
        def _():
            o_ref[...] += contrib


def kernel(x, Wr, temp, Wg, Wu, Wd):
    xf = x.reshape(_T, _D)
    t2 = temp.reshape(1, 1)
    s11 = jax.ShapeDtypeStruct((1, 1), jnp.float32)
    out, aux, ent, ovf = pl.pallas_call(
        _moe_kernel,
        grid=(_E, _HSPLIT),
        in_specs=[
            pl.BlockSpec((_T, _D), lambda e, h: (0, 0)),
            pl.BlockSpec((_E, _D), lambda e, h: (0, 0)),
            pl.BlockSpec((1, 1), lambda e, h: (0, 0)),
            pl.BlockSpec((1, _HB, _D), lambda e, h: (e, h, 0)),
            pl.BlockSpec((1, _HB, _D), lambda e, h: (e, h, 0)),
            pl.BlockSpec((1, _D, _HB), lambda e, h: (e, 0, h)),
        ],
        out_specs=(
            pl.BlockSpec((_T, _D), lambda e, h: (0, 0)),
            pl.BlockSpec((1, 1), lambda e, h: (0, 0)),
            pl.BlockSpec((1, 1), lambda e, h: (0, 0)),
            pl.BlockSpec((1, 1), lambda e, h: (0, 0)),
        ),
        out_shape=(
            jax.ShapeDtypeStruct((_T, _D), jnp.float32),
            s11, s11, s11,
        ),
        scratch_shapes=[
            pltpu.VMEM((_T, _E), jnp.float32),
            pltpu.VMEM((_T, _E), jnp.float32),
            pltpu.VMEM((_C, _D), jnp.float32),
            pltpu.VMEM((_C, _D), jnp.float32),
        ],
        compiler_params=pltpu.CompilerParams(
            dimension_semantics=("arbitrary", "arbitrary")),
    )(xf, Wr, t2, Wg, Wu, Wd)

    return (out.reshape(_B, _S, _D), aux[0, 0], ent[0, 0], ovf[0, 0])


# speedup vs baseline: 1.1031x; 1.1031x over previous
import jax
import jax.numpy as jnp
from jax.experimental import pallas as pl
from jax.experimental.pallas import tpu as pltpu

_B, _S, _D = 1, 2048, 768
_E, _K = 8, 2
_H = _D * 2
_CF = 1.25
_AUX = 0.01
_T = _B * _S
_C = max(1, int(_T * _K / _E * _CF))


def _moe_kernel(x_ref, wr_ref, t_ref, wg_ref, wu_ref, wd_ref,
                o_ref, aux_ref, ent_ref, ovf_ref, pos_scr, cf_scr, xb_scr):
    e = pl.program_id(0)
    f32 = jnp.float32
    bfl = jnp.bfloat16

    @pl.when(e == 0)
    def _cast_x():
        xb_scr[...] = x_ref[...].astype(bfl)

    @pl.when(e == 0)
    def _router():
        xf = x_ref[...]
        wr = wr_ref[...]
        t = jnp.clip(t_ref[0, 0], 0.1, 5.0)
        logits = jax.lax.dot_general(
            xf, wr, (((1,), (1,)), ((), ())),
            preferred_element_type=f32) / t
        m = jnp.max(logits, axis=1, keepdims=True)
        ex = jnp.exp(logits - m)
        probs = ex / jnp.sum(ex, axis=1, keepdims=True)

        eio = jax.lax.broadcasted_iota(jnp.int32, (_T, _E), 1)
        m0 = jnp.max(probs, axis=1, keepdims=True)
        i0 = jnp.min(jnp.where(probs == m0, eio, _E), axis=1, keepdims=True)
        p2 = jnp.where(eio == i0, -jnp.inf, probs)
        m1 = jnp.max(p2, axis=1, keepdims=True)
        i1 = jnp.min(jnp.where(p2 == m1, eio, _E), axis=1, keepdims=True)
        mask = ((eio == i0) | (eio == i1)).astype(f32)

        tio = jax.lax.broadcasted_iota(jnp.int32, (_T, _E), 0)
        cum = mask
        s = 1
        while s < _T:
            sh = pltpu.roll(cum, s, axis=0)
            cum = cum + jnp.where(tio >= s, sh, 0.0)
            s *= 2

        keep = mask * (cum <= _C).astype(f32)
        pos_scr[...] = jnp.where(keep > 0, cum - 1.0, -1.0)
        cf_scr[...] = probs * keep

        importance = jnp.sum(probs, axis=0, keepdims=True) / _T
        load = jnp.sum(mask, axis=0, keepdims=True) / (_T + 1e-06)
        aux_ref[...] = jnp.sum(importance * load, axis=1,
                               keepdims=True) * _E * _AUX
        plogp = probs * jnp.log(jnp.clip(probs, 1e-08))
        ent = -jnp.sum(plogp, axis=1, keepdims=True)
        ent_ref[...] = jnp.sum(ent, axis=0, keepdims=True) / _T * 0.01
        nm = jnp.sum(mask, axis=0, keepdims=True)
        nk = jnp.sum(keep, axis=0, keepdims=True)
        tot = jnp.sum(nm, axis=1, keepdims=True)
        drop = jnp.sum(nm - nk, axis=1, keepdims=True)
        ovf_ref[...] = drop / jnp.maximum(tot, 1.0)

    lio = jax.lax.broadcasted_iota(jnp.int32, (_T, _E), 1)
    ohm = (lio == e).astype(f32)
    pos_col = jnp.sum(pos_scr[...] * ohm, axis=1, keepdims=True)
    cf_col = jnp.sum(cf_scr[...] * ohm, axis=1, keepdims=True)

    cj = jax.lax.broadcasted_iota(jnp.int32, (_T, _C), 1)
    ptj = jnp.broadcast_to(pos_col, (_T, _C)).astype(jnp.int32)
    PT = (ptj == cj).astype(bfl)

    buf = jax.lax.dot_general(PT, xb_scr[...], (((0,), (0,)), ((), ())),
                              preferred_element_type=f32)
    wg = wg_ref[0]
    wu = wu_ref[0]
    wd = wd_ref[0]
    g = jax.lax.dot_general(buf, wg, (((1,), (1,)), ((), ())),
                            preferred_element_type=f32)
    u = jax.lax.dot_general(buf, wu, (((1,), (1,)), ((), ())),
                            preferred_element_type=f32)
    hid = (u * jax.nn.sigmoid(u)) * g
    ob = jax.lax.dot_general(hid, wd, (((1,), (1,)), ((), ())),
                             preferred_element_type=f32)
    comb = jax.lax.dot_general(PT, ob.astype(bfl), (((1,), (0,)), ((), ())),
                               preferred_element_type=f32)
    contrib = comb * cf_col

    @pl.when(e == 0)
    def _():
        o_ref[...] = contrib

    @pl.when(e != 0)
    def _():
        o_ref[...] += contrib


def kernel(x, Wr, temp, Wg, Wu, Wd):
    xf = x.reshape(_T, _D)
    t2 = temp.reshape(1, 1)
    s11 = jax.ShapeDtypeStruct((1, 1), jnp.float32)
    out, aux, ent, ovf = pl.pallas_call(
        _moe_kernel,
        grid=(_E,),
        in_specs=[
            pl.BlockSpec((_T, _D), lambda e: (0, 0)),
            pl.BlockSpec((_E, _D), lambda e: (0, 0)),
            pl.BlockSpec((1, 1), lambda e: (0, 0)),
            pl.BlockSpec((1, _H, _D), lambda e: (e, 0, 0)),
            pl.BlockSpec((1, _H, _D), lambda e: (e, 0, 0)),
            pl.BlockSpec((1, _D, _H), lambda e: (e, 0, 0)),
        ],
        out_specs=(
            pl.BlockSpec((_T, _D), lambda e: (0, 0)),
            pl.BlockSpec((1, 1), lambda e: (0, 0)),
            pl.BlockSpec((1, 1), lambda e: (0, 0)),
            pl.BlockSpec((1, 1), lambda e: (0, 0)),
        ),
        out_shape=(
            jax.ShapeDtypeStruct((_T, _D), jnp.float32),
            s11, s11, s11,
        ),
        scratch_shapes=[
            pltpu.VMEM((_T, _E), jnp.float32),
            pltpu.VMEM((_T, _E), jnp.float32),
            pltpu.VMEM((_T, _D), jnp.bfloat16),
        ],
        compiler_params=pltpu.CompilerParams(
            dimension_semantics=("arbitrary",)),
    )(xf, Wr, t2, Wg, Wu, Wd)

    return (out.reshape(_B, _S, _D), aux[0, 0], ent[0, 0], ovf[0, 0])


# speedup vs baseline: 1.1052x; 1.0019x over previous
import jax
import jax.numpy as jnp
from jax.experimental import pallas as pl
from jax.experimental.pallas import tpu as pltpu

_B, _S, _D = 1, 2048, 768
_E, _K = 8, 2
_H = _D * 2
_CF = 1.25
_AUX = 0.01
_T = _B * _S
_C = max(1, int(_T * _K / _E * _CF))


def _moe_kernel(x_ref, wr_ref, t_ref, wg_ref, wu_ref, wd_ref,
                o_ref, aux_ref, ent_ref, ovf_ref, pos_scr, cf_scr, xb_scr):
    e = pl.program_id(0)
    f32 = jnp.float32
    bfl = jnp.bfloat16

    @pl.when(e == 0)
    def _cast_x():
        xb_scr[...] = x_ref[...].astype(bfl)

    @pl.when(e == 0)
    def _router():
        xf = x_ref[...]
        wr = wr_ref[...]
        t = jnp.clip(t_ref[0, 0], 0.1, 5.0)
        logits = jax.lax.dot_general(
            xf, wr, (((1,), (1,)), ((), ())),
            preferred_element_type=f32) / t
        m = jnp.max(logits, axis=1, keepdims=True)
        ex = jnp.exp(logits - m)
        probs = ex / jnp.sum(ex, axis=1, keepdims=True)

        eio = jax.lax.broadcasted_iota(jnp.int32, (_T, _E), 1)
        m0 = jnp.max(probs, axis=1, keepdims=True)
        i0 = jnp.min(jnp.where(probs == m0, eio, _E), axis=1, keepdims=True)
        p2 = jnp.where(eio == i0, -jnp.inf, probs)
        m1 = jnp.max(p2, axis=1, keepdims=True)
        i1 = jnp.min(jnp.where(p2 == m1, eio, _E), axis=1, keepdims=True)
        mask = ((eio == i0) | (eio == i1)).astype(f32)

        tio = jax.lax.broadcasted_iota(jnp.int32, (_T, _E), 0)
        cum = mask
        s = 1
        while s < _T:
            sh = pltpu.roll(cum, s, axis=0)
            cum = cum + jnp.where(tio >= s, sh, 0.0)
            s *= 2

        keep = mask * (cum <= _C).astype(f32)
        pos_scr[...] = jnp.where(keep > 0, cum - 1.0, -1.0)
        cf_scr[...] = probs * keep

        importance = jnp.sum(probs, axis=0, keepdims=True) / _T
        load = jnp.sum(mask, axis=0, keepdims=True) / (_T + 1e-06)
        aux_ref[...] = jnp.sum(importance * load, axis=1,
                               keepdims=True) * _E * _AUX
        plogp = probs * jnp.log(jnp.clip(probs, 1e-08))
        ent = -jnp.sum(plogp, axis=1, keepdims=True)
        ent_ref[...] = jnp.sum(ent, axis=0, keepdims=True) / _T * 0.01
        nm = jnp.sum(mask, axis=0, keepdims=True)
        nk = jnp.sum(keep, axis=0, keepdims=True)
        tot = jnp.sum(nm, axis=1, keepdims=True)
        drop = jnp.sum(nm - nk, axis=1, keepdims=True)
        ovf_ref[...] = drop / jnp.maximum(tot, 1.0)

    lio = jax.lax.broadcasted_iota(jnp.int32, (_T, _E), 1)
    ohm = (lio == e).astype(f32)
    pos_col = jnp.sum(pos_scr[...] * ohm, axis=1, keepdims=True)
    cf_col = jnp.sum(cf_scr[...] * ohm, axis=1, keepdims=True)

    cj = jax.lax.broadcasted_iota(jnp.int32, (_T, _C), 1)
    ptj = jnp.broadcast_to(pos_col, (_T, _C)).astype(jnp.int32)
    PT = (ptj == cj).astype(bfl)

    buf = jax.lax.dot_general(PT, xb_scr[...], (((0,), (0,)), ((), ())),
                              preferred_element_type=f32)
    wg = wg_ref[0]
    wu = wu_ref[0]
    wd = wd_ref[0]
    g = jax.lax.dot_general(buf, wg, (((1,), (1,)), ((), ())),
                            preferred_element_type=f32)
    u = jax.lax.dot_general(buf, wu, (((1,), (1,)), ((), ())),
                            preferred_element_type=f32)
    hid = (u * jax.nn.sigmoid(u)) * g
    ob = jax.lax.dot_general(hid, wd, (((1,), (1,)), ((), ())),
                             preferred_element_type=f32)
    comb = jax.lax.dot_general(PT, ob.astype(bfl), (((1,), (0,)), ((), ())),
                               preferred_element_type=f32)
    contrib = comb * cf_col

    @pl.when(e == 0)
    def _():
        o_ref[...] = contrib

    @pl.when(e != 0)
    def _():
        o_ref[...] += contrib


def kernel(x, Wr, temp, Wg, Wu, Wd):
    xf = x.reshape(_T, _D)
    t2 = temp.reshape(1, 1)
    s11 = jax.ShapeDtypeStruct((1, 1), jnp.float32)
    out, aux, ent, ovf = pl.pallas_call(
        _moe_kernel,
        grid=(_E,),
        in_specs=[
            pl.BlockSpec((_T, _D), lambda e: (0, 0)),
            pl.BlockSpec((_E, _D), lambda e: (0, 0)),
            pl.BlockSpec((1, 1), lambda e: (0, 0)),
            pl.BlockSpec((1, _H, _D), lambda e: (e, 0, 0)),
            pl.BlockSpec((1, _H, _D), lambda e: (e, 0, 0)),
            pl.BlockSpec((1, _D, _H), lambda e: (e, 0, 0)),
        ],
        out_specs=(
            pl.BlockSpec((_T, _D), lambda e: (0, 0)),
            pl.BlockSpec((1, 1), lambda e: (0, 0)),
            pl.BlockSpec((1, 1), lambda e: (0, 0)),
            pl.BlockSpec((1, 1), lambda e: (0, 0)),
        ),
        out_shape=(
            jax.ShapeDtypeStruct((_T, _D), jnp.float32),
            s11, s11, s11,
        ),
        scratch_shapes=[
            pltpu.VMEM((_T, _E), jnp.float32),
            pltpu.VMEM((_T, _E), jnp.float32),
            pltpu.VMEM((_T, _D), jnp.bfloat16),
        ],
        compiler_params=pltpu.CompilerParams(
            dimension_semantics=("arbitrary",),
            vmem_limit_bytes=128 * 1024 * 1024),
    )(xf, Wr, t2, Wg, Wu, Wd)

    return (out.reshape(_B, _S, _D), aux[0, 0], ent[0, 0], ovf[0, 0])
